# baseline (device time: 103043 ns/iter reference)
import jax
import jax.numpy as jnp
from jax import lax
from jax.experimental import pallas as pl
from jax.experimental.pallas import tpu as pltpu

N_DEV = 16
B_LOC = 2
SQ = 256
SKV = 256
HQ_LOC = 4
DH = 64
D_MODEL = 512
BLK = 64


def kernel(x, Wq, K_ext, V_ext, Wo):
    i = lax.axis_index("i")
    xb = x.astype(jnp.bfloat16)
    wq = (Wq * 0.125).astype(jnp.bfloat16)
    wo = Wo.astype(jnp.bfloat16)
    k_loc = lax.dynamic_slice_in_dim(K_ext, i * B_LOC, B_LOC, axis=0)
    v_loc = lax.dynamic_slice_in_dim(V_ext, i * B_LOC, B_LOC, axis=0)
    k_loc = jnp.transpose(k_loc, (2, 0, 1, 3)).astype(jnp.bfloat16)
    v_loc = jnp.transpose(v_loc, (2, 0, 1, 3)).astype(jnp.bfloat16)


    def body(x_ref, wq_ref, wo_ref, k_ref, v_ref, out_ref,
             wq_comm, wo_comm, wq_send, wq_recv, wo_send, wo_recv):
        my = lax.axis_index("i")
        right = lax.rem(my + 1, N_DEV)
        left = lax.rem(my + N_DEV - 1, N_DEV)

        barrier = pltpu.get_barrier_semaphore()
        pl.semaphore_signal(barrier, inc=1, device_id=(left,),
                            device_id_type=pl.DeviceIdType.MESH)
        pl.semaphore_signal(barrier, inc=1, device_id=(right,),
                            device_id_type=pl.DeviceIdType.MESH)
        pl.semaphore_wait(barrier, 2)

        out_ref[...] = jnp.zeros((B_LOC, SQ, D_MODEL), jnp.float32)
        wq_comm[0] = wq_ref[...]
        wo_comm[0] = wo_ref[...]

        def dot_t(a, bm):
            return lax.dot_general(a, bm, (((1,), (1,)), ((), ())),
                                   preferred_element_type=jnp.float32)

        def softmax_ctx(ss, vs):
            us = [jnp.exp(s) for s in ss]
            r = sum(jnp.sum(u, axis=-1, keepdims=True) for u in us)
            c = sum(jnp.dot(u.astype(jnp.bfloat16), v,
                            preferred_element_type=jnp.float32)
                    for u, v in zip(us, vs))
            return (c * (1.0 / r)).astype(jnp.bfloat16)

        def compute(slot, j):
            wq_s = wq_comm[slot]
            wo_s = wo_comm[slot]
            for b in range(B_LOC):
                q_all = jnp.dot(x_ref[b], wq_s,
                                preferred_element_type=jnp.float32)
                q_all = q_all.astype(jnp.bfloat16)
                acc0 = jnp.zeros((64, D_MODEL), jnp.float32)
                accb = jnp.zeros((128, D_MODEL), jnp.float32)
                acc3 = jnp.zeros((64, D_MODEL), jnp.float32)
                for hh in range(HQ_LOC):
                    gh = j * HQ_LOC + hh
                    sl = slice(hh * DH, (hh + 1) * DH)
                    k0 = k_ref[gh, b, 0:64]
                    k3 = k_ref[gh, b, 192:256]
                    kb_ = k_ref[gh, b, 0:192]
                    v0 = v_ref[gh, b, 0:64]
                    v3 = v_ref[gh, b, 192:256]
                    vb = v_ref[gh, b, 0:192]
                    q0 = q_all[0:64, sl]
                    q3 = q_all[192:256, sl]
                    qb_ = q_all[64:192, sl]
                    c0 = softmax_ctx([dot_t(q0, k0), dot_t(q0, k3)],
                                     [v0, v3])
                    c3 = softmax_ctx([dot_t(q3, k0), dot_t(q3, k3)],
                                     [v0, v3])
                    cb = softmax_ctx([dot_t(qb_, kb_)], [vb])
                    wo_h = wo_s[sl]
                    acc0 = acc0 + jnp.dot(
                        c0, wo_h, preferred_element_type=jnp.float32)
                    accb = accb + jnp.dot(
                        cb, wo_h, preferred_element_type=jnp.float32)
                    acc3 = acc3 + jnp.dot(
                        c3, wo_h, preferred_element_type=jnp.float32)
                out_ref[b, 0:64] += acc0
                out_ref[b, 64:192] += accb
                out_ref[b, 192:256] += acc3

        def mk(to_right, src, dst, ssem, rsem):
            tgt = lax.select(to_right, right, left)
            rq = pltpu.make_async_remote_copy(
                src_ref=wq_comm.at[src], dst_ref=wq_comm.at[dst],
                send_sem=wq_send.at[ssem], recv_sem=wq_recv.at[rsem],
                device_id=(tgt,), device_id_type=pl.DeviceIdType.MESH)
            ro = pltpu.make_async_remote_copy(
                src_ref=wo_comm.at[src], dst_ref=wo_comm.at[dst],
                send_sem=wo_send.at[ssem], recv_sem=wo_recv.at[rsem],
                device_id=(tgt,), device_id_type=pl.DeviceIdType.MESH)
            return rq, ro

        def start(descs):
            descs[0].start()
            descs[1].start()

        def wait(descs):
            descs[0].wait()
            descs[1].wait()

        t_one = jnp.bool_(True)
        t_zero = jnp.bool_(False)
        start(mk(t_one, 0, 1, 0, 1))
        start(mk(t_zero, 0, 9, 8, 9))
        compute(0, my)

        def step(s, carry):
            wait(mk(t_one, s - 1, s, s - 1, s))

            @pl.when(s < 8)
            def _():
                start(mk(t_one, s, s + 1, s, s + 1))

            @pl.when(s <= 7)
            def _():
                src = jnp.where(s == 1, 0, 8 + s - 1)
                wait(mk(t_zero, src, 8 + s, 8 + s - 1, 8 + s))

                @pl.when(s < 7)
                def _():
                    start(mk(t_zero, 8 + s, 8 + s + 1, 8 + s, 8 + s + 1))

            compute(s, lax.rem(my - s + N_DEV, N_DEV))

            @pl.when(s <= 7)
            def _():
                compute(8 + s, lax.rem(my + s, N_DEV))
            return carry

        lax.fori_loop(1, 9, step, 0)

    return pl.pallas_call(
        body,
        out_shape=jax.ShapeDtypeStruct((B_LOC, SQ, D_MODEL), jnp.float32),
        in_specs=[pl.BlockSpec(memory_space=pltpu.VMEM)] * 5,
        out_specs=pl.BlockSpec(memory_space=pltpu.VMEM),
        scratch_shapes=[
            pltpu.VMEM((N_DEV, D_MODEL, HQ_LOC * DH), jnp.bfloat16),
            pltpu.VMEM((N_DEV, HQ_LOC * DH, D_MODEL), jnp.bfloat16),
            pltpu.SemaphoreType.DMA((N_DEV,)),
            pltpu.SemaphoreType.DMA((N_DEV,)),
            pltpu.SemaphoreType.DMA((N_DEV,)),
            pltpu.SemaphoreType.DMA((N_DEV,)),
        ],
        compiler_params=pltpu.CompilerParams(collective_id=0),
    )(xb, wq, wo, k_loc, v_loc)


# device time: 83735 ns/iter; 1.2306x vs baseline; 1.2306x over previous
import jax
import jax.numpy as jnp
from jax import lax
from jax.experimental import pallas as pl
from jax.experimental.pallas import tpu as pltpu

N_DEV = 16
B_LOC = 2
SQ = 256
SKV = 256
HQ_LOC = 4
DH = 64
D_MODEL = 512
BLK = 64


def kernel(x, Wq, K_ext, V_ext, Wo):
    i = lax.axis_index("i")
    xb = x.astype(jnp.bfloat16)
    wq = (Wq * 0.125).astype(jnp.bfloat16)
    wo = Wo.astype(jnp.bfloat16)
    k_loc = lax.dynamic_slice_in_dim(K_ext, i * B_LOC, B_LOC, axis=0)
    v_loc = lax.dynamic_slice_in_dim(V_ext, i * B_LOC, B_LOC, axis=0)
    k_loc = jnp.transpose(k_loc, (2, 0, 1, 3)).astype(jnp.bfloat16)
    v_loc = jnp.transpose(v_loc, (2, 0, 1, 3)).astype(jnp.bfloat16)


    def body(x_ref, wq_ref, wo_ref, k_ref, v_ref, out_ref,
             wq_comm, wo_comm, wq_send, wq_recv, wo_send, wo_recv):
        my = lax.axis_index("i")
        right = lax.rem(my + 1, N_DEV)
        left = lax.rem(my + N_DEV - 1, N_DEV)

        barrier = pltpu.get_barrier_semaphore()
        pl.semaphore_signal(barrier, inc=1, device_id=(left,),
                            device_id_type=pl.DeviceIdType.MESH)
        pl.semaphore_signal(barrier, inc=1, device_id=(right,),
                            device_id_type=pl.DeviceIdType.MESH)
        pl.semaphore_wait(barrier, 2)

        qb = lax.broadcasted_iota(jnp.int32, (SQ, SKV), 0) // BLK
        kb = lax.broadcasted_iota(jnp.int32, (SQ, SKV), 1) // BLK
        keep = (qb == kb) | (kb == 0) | (lax.rem(qb + kb, 3) == 0)
        bias = jnp.where(keep, 0.0, -1e9).astype(jnp.float32)

        out_ref[...] = jnp.zeros((B_LOC, SQ, D_MODEL), jnp.float32)
        wq_comm[0] = wq_ref[...]
        wo_comm[0] = wo_ref[...]

        def compute(slot, j):
            wq_s = wq_comm[slot]
            wo_s = wo_comm[slot]
            for b in range(B_LOC):
                q_all = jnp.dot(x_ref[b], wq_s,
                                preferred_element_type=jnp.float32)
                q_all = q_all.astype(jnp.bfloat16)
                ctxs = []
                for hh in range(HQ_LOC):
                    gh = j * HQ_LOC + hh
                    q = q_all[:, hh * DH:(hh + 1) * DH]
                    s = lax.dot_general(
                        q, k_ref[gh, b], (((1,), (1,)), ((), ())),
                        preferred_element_type=jnp.float32) + bias
                    u = jnp.exp(s)
                    r = 1.0 / jnp.sum(u, axis=-1, keepdims=True)
                    ctx = jnp.dot(u.astype(jnp.bfloat16), v_ref[gh, b],
                                  preferred_element_type=jnp.float32) * r
                    ctxs.append(ctx.astype(jnp.bfloat16))
                ctx_full = jnp.concatenate(ctxs, axis=1)
                out_ref[b] += jnp.dot(ctx_full, wo_s,
                                      preferred_element_type=jnp.float32)

        def mk(to_right, src, dst, ssem, rsem):
            tgt = lax.select(to_right, right, left)
            rq = pltpu.make_async_remote_copy(
                src_ref=wq_comm.at[src], dst_ref=wq_comm.at[dst],
                send_sem=wq_send.at[ssem], recv_sem=wq_recv.at[rsem],
                device_id=(tgt,), device_id_type=pl.DeviceIdType.MESH)
            ro = pltpu.make_async_remote_copy(
                src_ref=wo_comm.at[src], dst_ref=wo_comm.at[dst],
                send_sem=wo_send.at[ssem], recv_sem=wo_recv.at[rsem],
                device_id=(tgt,), device_id_type=pl.DeviceIdType.MESH)
            return rq, ro

        def start(descs):
            descs[0].start()
            descs[1].start()

        def wait(descs):
            descs[0].wait()
            descs[1].wait()

        t_one = jnp.bool_(True)
        t_zero = jnp.bool_(False)
        start(mk(t_one, 0, 1, 0, 1))
        start(mk(t_zero, 0, 9, 8, 9))
        compute(0, my)

        def step(s, carry):
            wait(mk(t_one, s - 1, s, s - 1, s))

            @pl.when(s < 8)
            def _():
                start(mk(t_one, s, s + 1, s, s + 1))

            @pl.when(s <= 7)
            def _():
                src = jnp.where(s == 1, 0, 8 + s - 1)
                wait(mk(t_zero, src, 8 + s, 8 + s - 1, 8 + s))

                @pl.when(s < 7)
                def _():
                    start(mk(t_zero, 8 + s, 8 + s + 1, 8 + s, 8 + s + 1))

            compute(s, lax.rem(my - s + N_DEV, N_DEV))

            @pl.when(s <= 7)
            def _():
                compute(8 + s, lax.rem(my + s, N_DEV))
            return carry

        lax.fori_loop(1, 9, step, 0)

    return pl.pallas_call(
        body,
        out_shape=jax.ShapeDtypeStruct((B_LOC, SQ, D_MODEL), jnp.float32),
        in_specs=[pl.BlockSpec(memory_space=pltpu.VMEM)] * 5,
        out_specs=pl.BlockSpec(memory_space=pltpu.VMEM),
        scratch_shapes=[
            pltpu.VMEM((N_DEV, D_MODEL, HQ_LOC * DH), jnp.bfloat16),
            pltpu.VMEM((N_DEV, HQ_LOC * DH, D_MODEL), jnp.bfloat16),
            pltpu.SemaphoreType.DMA((N_DEV,)),
            pltpu.SemaphoreType.DMA((N_DEV,)),
            pltpu.SemaphoreType.DMA((N_DEV,)),
            pltpu.SemaphoreType.DMA((N_DEV,)),
        ],
        compiler_params=pltpu.CompilerParams(collective_id=0),
    )(xb, wq, wo, k_loc, v_loc)
